# baseline (speedup 1.0000x reference)
import functools

import jax
import jax.numpy as jnp
from jax import lax
from jax.experimental import pallas as pl
from jax.experimental.pallas import tpu as pltpu
from jax.experimental.pallas import tpu_sc as plsc

BATCH = 16384
LATENT_DIM = 16
NUM_CORES = 2
NUM_SUBCORES = 16
NUM_WORKERS = NUM_CORES * NUM_SUBCORES
ROWS_PER_WORKER = BATCH // NUM_WORKERS
IDX_CHUNK = 128
NUM_CHUNKS = ROWS_PER_WORKER // IDX_CHUNK
GROUPS = ROWS_PER_WORKER // LATENT_DIM


def _gmf_body(uidx_hbm, iidx_hbm, emb_ut_hbm, emb_it_hbm, w_hbm, b_hbm,
              out_hbm, uidx_v, iidx_v, u_gath, i_gath, w_v, b_v, out_v, sem):
    wid = lax.axis_index("c") * NUM_SUBCORES + lax.axis_index("s")

    pltpu.sync_copy(uidx_hbm.at[wid], uidx_v)
    pltpu.sync_copy(iidx_hbm.at[wid], iidx_v)
    pltpu.sync_copy(w_hbm, w_v)
    pltpu.sync_copy(b_hbm, b_v)

    descs = []
    for d in range(LATENT_DIM):
        for j in range(NUM_CHUNKS):
            dst = pl.ds(j * IDX_CHUNK, IDX_CHUNK)
            descs.append(pltpu.async_copy(
                emb_ut_hbm.at[d].at[uidx_v.at[j]], u_gath.at[d].at[dst], sem))
            descs.append(pltpu.async_copy(
                emb_it_hbm.at[d].at[iidx_v.at[j]], i_gath.at[d].at[dst], sem))
    for dsc in descs:
        dsc.wait()

    b_reg = b_v[...]
    w_reg = w_v[...]

    def group(g, carry):
        sl = pl.ds(g * 16, 16)
        acc = jnp.zeros((16,), jnp.float32)
        for d in range(LATENT_DIM):
            acc = acc + u_gath[d, sl] * i_gath[d, sl] * w_reg[d]
        logits = acc + b_reg
        rating = 1.0 / (1.0 + jnp.exp(-logits))
        out_v[sl] = rating
        return carry

    lax.fori_loop(0, GROUPS, group, 0)

    pltpu.sync_copy(out_v, out_hbm.at[pl.ds(wid * ROWS_PER_WORKER, ROWS_PER_WORKER)])


@jax.jit
def _gmf(uidx3, iidx3, emb_ut, emb_it, w16, b16):
    mesh = plsc.VectorSubcoreMesh(core_axis_name="c", subcore_axis_name="s")
    f = functools.partial(
        pl.kernel,
        mesh=mesh,
        out_type=jax.ShapeDtypeStruct((BATCH,), jnp.float32),
        compiler_params=pltpu.CompilerParams(
            needs_layout_passes=False, use_tc_tiling_on_sc=False),
        scratch_types=[
            pltpu.VMEM((NUM_CHUNKS, IDX_CHUNK), jnp.int32),
            pltpu.VMEM((NUM_CHUNKS, IDX_CHUNK), jnp.int32),
            pltpu.VMEM((LATENT_DIM, ROWS_PER_WORKER), jnp.float32),
            pltpu.VMEM((LATENT_DIM, ROWS_PER_WORKER), jnp.float32),
            pltpu.VMEM((LATENT_DIM,), jnp.float32),
            pltpu.VMEM((LATENT_DIM,), jnp.float32),
            pltpu.VMEM((ROWS_PER_WORKER,), jnp.float32),
            pltpu.SemaphoreType.DMA,
        ],
    )(_gmf_body)
    return f(uidx3, iidx3, emb_ut, emb_it, w16, b16)


def kernel(user_indices, item_indices, domain_idc, embedding_user,
           embedding_item, affine_W, affine_b):
    del domain_idc
    uidx3 = user_indices.reshape(NUM_WORKERS, NUM_CHUNKS, IDX_CHUNK)
    iidx3 = item_indices.reshape(NUM_WORKERS, NUM_CHUNKS, IDX_CHUNK)
    emb_ut = embedding_user.T
    emb_it = embedding_item.T
    w16 = affine_W.reshape(LATENT_DIM)
    b16 = jnp.broadcast_to(affine_b, (LATENT_DIM,))
    out = _gmf(uidx3, iidx3, emb_ut, emb_it, w16, b16)
    return out.reshape(BATCH, 1)


# speedup vs baseline: 2.4438x; 2.4438x over previous
import functools

import jax
import jax.numpy as jnp
from jax import lax
from jax.experimental import pallas as pl
from jax.experimental.pallas import tpu as pltpu
from jax.experimental.pallas import tpu_sc as plsc

BATCH = 16384
LATENT_DIM = 16
TABLE_ROWS = 1000000
LANES = 128
SLAB_TILES = 7816
SLAB = SLAB_TILES * LANES
NUM_CORES = 2
NUM_SUBCORES = 16
NUM_WORKERS = NUM_CORES * NUM_SUBCORES
ROWS_PER_WORKER = BATCH // NUM_WORKERS
IDX_CHUNK = 128
NUM_CHUNKS = ROWS_PER_WORKER // IDX_CHUNK
GROUPS = ROWS_PER_WORKER // LATENT_DIM
FLAT_TILES = 8
FLAT_GRID = SLAB_TILES // FLAT_TILES


def _flatten_body(in_ref, out_ref):
    out_ref[...] = in_ref[...].reshape(LATENT_DIM, FLAT_TILES, LANES)


@jax.jit
def _flatten(embT):
    return pl.pallas_call(
        _flatten_body,
        grid=(FLAT_GRID,),
        in_specs=[pl.BlockSpec((LATENT_DIM, FLAT_TILES * LANES), lambda j: (0, j))],
        out_specs=pl.BlockSpec((LATENT_DIM, FLAT_TILES, LANES), lambda j: (0, j, 0)),
        out_shape=jax.ShapeDtypeStruct((LATENT_DIM, SLAB_TILES, LANES), jnp.float32),
        compiler_params=pltpu.CompilerParams(
            dimension_semantics=("arbitrary",)),
    )(embT)


def _gmf_body(uidx_hbm, iidx_hbm, uflat_hbm, iflat_hbm, w_hbm, b_hbm,
              out_hbm, uidx_v, iidx_v, u_gath, i_gath, w_v, b_v, out_v, sem):
    wid = lax.axis_index("c") * NUM_SUBCORES + lax.axis_index("s")

    pltpu.sync_copy(uidx_hbm.at[wid], uidx_v)
    pltpu.sync_copy(iidx_hbm.at[wid], iidx_v)
    pltpu.sync_copy(w_hbm, w_v)
    pltpu.sync_copy(b_hbm, b_v)

    descs = []
    for d in range(LATENT_DIM):
        u_stripe = uflat_hbm.at[pl.ds(d * SLAB, SLAB)]
        i_stripe = iflat_hbm.at[pl.ds(d * SLAB, SLAB)]
        for j in range(NUM_CHUNKS):
            dst = pl.ds(j * IDX_CHUNK, IDX_CHUNK)
            descs.append(pltpu.async_copy(
                u_stripe.at[uidx_v.at[j]], u_gath.at[d].at[dst], sem))
            descs.append(pltpu.async_copy(
                i_stripe.at[iidx_v.at[j]], i_gath.at[d].at[dst], sem))
    for dsc in descs:
        dsc.wait()

    b_reg = b_v[...]
    w_reg = w_v[...]

    def group(g, carry):
        sl = pl.ds(g * 16, 16)
        acc = jnp.zeros((16,), jnp.float32)
        for d in range(LATENT_DIM):
            acc = acc + u_gath[d, sl] * i_gath[d, sl] * w_reg[d]
        logits = acc + b_reg
        rating = 1.0 / (1.0 + jnp.exp(-logits))
        out_v[sl] = rating
        return carry

    lax.fori_loop(0, GROUPS, group, 0)

    pltpu.sync_copy(out_v, out_hbm.at[pl.ds(wid * ROWS_PER_WORKER, ROWS_PER_WORKER)])


@jax.jit
def _gmf(uidx3, iidx3, uflat, iflat, w16, b16):
    mesh = plsc.VectorSubcoreMesh(core_axis_name="c", subcore_axis_name="s")
    f = functools.partial(
        pl.kernel,
        mesh=mesh,
        out_type=jax.ShapeDtypeStruct((BATCH,), jnp.float32),
        compiler_params=pltpu.CompilerParams(
            needs_layout_passes=False, use_tc_tiling_on_sc=False),
        scratch_types=[
            pltpu.VMEM((NUM_CHUNKS, IDX_CHUNK), jnp.int32),
            pltpu.VMEM((NUM_CHUNKS, IDX_CHUNK), jnp.int32),
            pltpu.VMEM((LATENT_DIM, ROWS_PER_WORKER), jnp.float32),
            pltpu.VMEM((LATENT_DIM, ROWS_PER_WORKER), jnp.float32),
            pltpu.VMEM((LATENT_DIM,), jnp.float32),
            pltpu.VMEM((LATENT_DIM,), jnp.float32),
            pltpu.VMEM((ROWS_PER_WORKER,), jnp.float32),
            pltpu.SemaphoreType.DMA,
        ],
    )(_gmf_body)
    return f(uidx3, iidx3, uflat, iflat, w16, b16)


def kernel(user_indices, item_indices, domain_idc, embedding_user,
           embedding_item, affine_W, affine_b):
    del domain_idc
    uidx3 = user_indices.reshape(NUM_WORKERS, NUM_CHUNKS, IDX_CHUNK)
    iidx3 = item_indices.reshape(NUM_WORKERS, NUM_CHUNKS, IDX_CHUNK)
    uflat = _flatten(embedding_user.T).reshape(LATENT_DIM * SLAB)
    iflat = _flatten(embedding_item.T).reshape(LATENT_DIM * SLAB)
    w16 = affine_W.reshape(LATENT_DIM)
    b16 = jnp.broadcast_to(affine_b, (LATENT_DIM,))
    out = _gmf(uidx3, iidx3, uflat, iflat, w16, b16)
    return out.reshape(BATCH, 1)


# speedup vs baseline: 20.6168x; 8.4364x over previous
import functools

import jax
import jax.numpy as jnp
from jax import lax
from jax.experimental import pallas as pl
from jax.experimental.pallas import tpu as pltpu
from jax.experimental.pallas import tpu_sc as plsc

BATCH = 16384
LATENT_DIM = 16
TABLE_ROWS = 1000000
LANES = 128
SLAB_TILES = 7840
SLAB = SLAB_TILES * LANES
NUM_CORES = 2
NUM_SUBCORES = 16
NUM_WORKERS = NUM_CORES * NUM_SUBCORES
ROWS_PER_WORKER = BATCH // NUM_WORKERS
IDX_CHUNK = 128
NUM_CHUNKS = ROWS_PER_WORKER // IDX_CHUNK
GROUPS = ROWS_PER_WORKER // LATENT_DIM
FLAT_TILES = 560
FLAT_GRID = SLAB_TILES // FLAT_TILES


def _flatten_body(in_ref, out_ref):
    out_ref[...] = in_ref[...].reshape(LATENT_DIM, FLAT_TILES, LANES)


@jax.jit
def _flatten(embT):
    return pl.pallas_call(
        _flatten_body,
        grid=(FLAT_GRID,),
        in_specs=[pl.BlockSpec((LATENT_DIM, FLAT_TILES * LANES), lambda j: (0, j))],
        out_specs=pl.BlockSpec((LATENT_DIM, FLAT_TILES, LANES), lambda j: (0, j, 0)),
        out_shape=jax.ShapeDtypeStruct((LATENT_DIM, SLAB_TILES, LANES), jnp.float32),
        compiler_params=pltpu.CompilerParams(
            dimension_semantics=("arbitrary",)),
    )(embT)


def _gmf_body(uidx_hbm, iidx_hbm, uflat_hbm, iflat_hbm, w_hbm, b_hbm,
              out_hbm, uidx_v, iidx_v, u_gath, i_gath, w_v, b_v, out_v, sem):
    wid = lax.axis_index("c") * NUM_SUBCORES + lax.axis_index("s")

    pltpu.sync_copy(uidx_hbm.at[wid], uidx_v)
    pltpu.sync_copy(iidx_hbm.at[wid], iidx_v)
    pltpu.sync_copy(w_hbm, w_v)
    pltpu.sync_copy(b_hbm, b_v)

    descs = []
    for d in range(LATENT_DIM):
        u_stripe = uflat_hbm.at[pl.ds(d * SLAB, SLAB)]
        i_stripe = iflat_hbm.at[pl.ds(d * SLAB, SLAB)]
        for j in range(NUM_CHUNKS):
            dst = pl.ds(j * IDX_CHUNK, IDX_CHUNK)
            descs.append(pltpu.async_copy(
                u_stripe.at[uidx_v.at[j]], u_gath.at[d].at[dst], sem))
            descs.append(pltpu.async_copy(
                i_stripe.at[iidx_v.at[j]], i_gath.at[d].at[dst], sem))
    for dsc in descs:
        dsc.wait()

    b_reg = b_v[...]
    w_reg = w_v[...]

    def group(g, carry):
        sl = pl.ds(g * 16, 16)
        acc = jnp.zeros((16,), jnp.float32)
        for d in range(LATENT_DIM):
            acc = acc + u_gath[d, sl] * i_gath[d, sl] * w_reg[d]
        logits = acc + b_reg
        rating = 1.0 / (1.0 + jnp.exp(-logits))
        out_v[sl] = rating
        return carry

    lax.fori_loop(0, GROUPS, group, 0)

    pltpu.sync_copy(out_v, out_hbm.at[pl.ds(wid * ROWS_PER_WORKER, ROWS_PER_WORKER)])


@jax.jit
def _gmf(uidx3, iidx3, uflat, iflat, w16, b16):
    mesh = plsc.VectorSubcoreMesh(core_axis_name="c", subcore_axis_name="s")
    f = functools.partial(
        pl.kernel,
        mesh=mesh,
        out_type=jax.ShapeDtypeStruct((BATCH,), jnp.float32),
        compiler_params=pltpu.CompilerParams(
            needs_layout_passes=False, use_tc_tiling_on_sc=False),
        scratch_types=[
            pltpu.VMEM((NUM_CHUNKS, IDX_CHUNK), jnp.int32),
            pltpu.VMEM((NUM_CHUNKS, IDX_CHUNK), jnp.int32),
            pltpu.VMEM((LATENT_DIM, ROWS_PER_WORKER), jnp.float32),
            pltpu.VMEM((LATENT_DIM, ROWS_PER_WORKER), jnp.float32),
            pltpu.VMEM((LATENT_DIM,), jnp.float32),
            pltpu.VMEM((LATENT_DIM,), jnp.float32),
            pltpu.VMEM((ROWS_PER_WORKER,), jnp.float32),
            pltpu.SemaphoreType.DMA,
        ],
    )(_gmf_body)
    return f(uidx3, iidx3, uflat, iflat, w16, b16)


def kernel(user_indices, item_indices, domain_idc, embedding_user,
           embedding_item, affine_W, affine_b):
    del domain_idc
    uidx3 = user_indices.reshape(NUM_WORKERS, NUM_CHUNKS, IDX_CHUNK)
    iidx3 = item_indices.reshape(NUM_WORKERS, NUM_CHUNKS, IDX_CHUNK)
    uflat = _flatten(embedding_user.T).reshape(LATENT_DIM * SLAB)
    iflat = _flatten(embedding_item.T).reshape(LATENT_DIM * SLAB)
    w16 = affine_W.reshape(LATENT_DIM)
    b16 = jnp.broadcast_to(affine_b, (LATENT_DIM,))
    out = _gmf(uidx3, iidx3, uflat, iflat, w16, b16)
    return out.reshape(BATCH, 1)


# speedup vs baseline: 21.3716x; 1.0366x over previous
import functools

import jax
import jax.numpy as jnp
from jax import lax
from jax.experimental import pallas as pl
from jax.experimental.pallas import tpu as pltpu
from jax.experimental.pallas import tpu_sc as plsc

BATCH = 16384
LATENT_DIM = 16
TABLE_ROWS = 1000000
LANES = 128
SLAB_TILES = 7840
SLAB = SLAB_TILES * LANES
NUM_CORES = 2
NUM_SUBCORES = 16
NUM_WORKERS = NUM_CORES * NUM_SUBCORES
ROWS_PER_WORKER = BATCH // NUM_WORKERS
IDX_CHUNK = 128
NUM_CHUNKS = ROWS_PER_WORKER // IDX_CHUNK
GROUPS = ROWS_PER_WORKER // LATENT_DIM
FLAT_TILES = 560
FLAT_GRID = SLAB_TILES // FLAT_TILES


def _flatten_body(u_ref, i_ref, uo_ref, io_ref):
    uo_ref[...] = u_ref[...].reshape(LATENT_DIM, FLAT_TILES, LANES)
    io_ref[...] = i_ref[...].reshape(LATENT_DIM, FLAT_TILES, LANES)


@jax.jit
def _flatten(emb_ut, emb_it):
    spec_in = pl.BlockSpec((LATENT_DIM, FLAT_TILES * LANES), lambda j: (0, j))
    spec_out = pl.BlockSpec((LATENT_DIM, FLAT_TILES, LANES), lambda j: (0, j, 0))
    shp = jax.ShapeDtypeStruct((LATENT_DIM, SLAB_TILES, LANES), jnp.float32)
    return pl.pallas_call(
        _flatten_body,
        grid=(FLAT_GRID,),
        in_specs=[spec_in, spec_in],
        out_specs=[spec_out, spec_out],
        out_shape=[shp, shp],
        compiler_params=pltpu.CompilerParams(
            dimension_semantics=("arbitrary",)),
    )(emb_ut, emb_it)


def _gmf_body(uidx_hbm, iidx_hbm, uflat_hbm, iflat_hbm, w_hbm, b_hbm,
              out_hbm, uidx_v, iidx_v, u_gath, i_gath, w_v, b_v, out_v, sem):
    wid = lax.axis_index("c") * NUM_SUBCORES + lax.axis_index("s")

    pltpu.sync_copy(uidx_hbm.at[wid], uidx_v)
    pltpu.sync_copy(iidx_hbm.at[wid], iidx_v)
    pltpu.sync_copy(w_hbm, w_v)
    pltpu.sync_copy(b_hbm, b_v)

    descs = []
    for d in range(LATENT_DIM):
        u_stripe = uflat_hbm.at[pl.ds(d * SLAB, SLAB)]
        i_stripe = iflat_hbm.at[pl.ds(d * SLAB, SLAB)]
        for j in range(NUM_CHUNKS):
            dst = pl.ds(j * IDX_CHUNK, IDX_CHUNK)
            descs.append(pltpu.async_copy(
                u_stripe.at[uidx_v.at[j]], u_gath.at[d].at[dst], sem))
            descs.append(pltpu.async_copy(
                i_stripe.at[iidx_v.at[j]], i_gath.at[d].at[dst], sem))
    for dsc in descs:
        dsc.wait()

    b_reg = b_v[...]
    w_reg = w_v[...]

    def group(g, carry):
        sl = pl.ds(g * 16, 16)
        acc = jnp.zeros((16,), jnp.float32)
        for d in range(LATENT_DIM):
            acc = acc + u_gath[d, sl] * i_gath[d, sl] * w_reg[d]
        logits = acc + b_reg
        rating = 1.0 / (1.0 + jnp.exp(-logits))
        out_v[sl] = rating
        return carry

    lax.fori_loop(0, GROUPS, group, 0)

    pltpu.sync_copy(out_v, out_hbm.at[pl.ds(wid * ROWS_PER_WORKER, ROWS_PER_WORKER)])


@jax.jit
def _gmf(uidx3, iidx3, uflat, iflat, w16, b16):
    mesh = plsc.VectorSubcoreMesh(core_axis_name="c", subcore_axis_name="s")
    f = functools.partial(
        pl.kernel,
        mesh=mesh,
        out_type=jax.ShapeDtypeStruct((BATCH,), jnp.float32),
        compiler_params=pltpu.CompilerParams(
            needs_layout_passes=False, use_tc_tiling_on_sc=False),
        scratch_types=[
            pltpu.VMEM((NUM_CHUNKS, IDX_CHUNK), jnp.int32),
            pltpu.VMEM((NUM_CHUNKS, IDX_CHUNK), jnp.int32),
            pltpu.VMEM((LATENT_DIM, ROWS_PER_WORKER), jnp.float32),
            pltpu.VMEM((LATENT_DIM, ROWS_PER_WORKER), jnp.float32),
            pltpu.VMEM((LATENT_DIM,), jnp.float32),
            pltpu.VMEM((LATENT_DIM,), jnp.float32),
            pltpu.VMEM((ROWS_PER_WORKER,), jnp.float32),
            pltpu.SemaphoreType.DMA,
        ],
    )(_gmf_body)
    return f(uidx3, iidx3, uflat, iflat, w16, b16)


def kernel(user_indices, item_indices, domain_idc, embedding_user,
           embedding_item, affine_W, affine_b):
    del domain_idc
    uidx3 = user_indices.reshape(NUM_WORKERS, NUM_CHUNKS, IDX_CHUNK)
    iidx3 = item_indices.reshape(NUM_WORKERS, NUM_CHUNKS, IDX_CHUNK)
    uslab, islab = _flatten(embedding_user.T, embedding_item.T)
    uflat = uslab.reshape(LATENT_DIM * SLAB)
    iflat = islab.reshape(LATENT_DIM * SLAB)
    w16 = affine_W.reshape(LATENT_DIM)
    b16 = jnp.broadcast_to(affine_b, (LATENT_DIM,))
    out = _gmf(uidx3, iidx3, uflat, iflat, w16, b16)
    return out.reshape(BATCH, 1)


# speedup vs baseline: 21.8152x; 1.0208x over previous
import functools

import jax
import jax.numpy as jnp
from jax import lax
from jax.experimental import pallas as pl
from jax.experimental.pallas import tpu as pltpu
from jax.experimental.pallas import tpu_sc as plsc

BATCH = 16384
LATENT_DIM = 16
TABLE_ROWS = 1000000
LANES = 128
SLAB_TILES = 7840
SLAB = SLAB_TILES * LANES
NUM_CORES = 2
NUM_SUBCORES = 16
NUM_WORKERS = NUM_CORES * NUM_SUBCORES
ROWS_PER_WORKER = BATCH // NUM_WORKERS
IDX_CHUNK = 128
NUM_CHUNKS = ROWS_PER_WORKER // IDX_CHUNK
GROUPS = ROWS_PER_WORKER // LATENT_DIM
GATH = LATENT_DIM * ROWS_PER_WORKER
FLAT_TILES = 1120
FLAT_GRID = SLAB_TILES // FLAT_TILES

_SC_PARAMS = pltpu.CompilerParams(
    needs_layout_passes=False, use_tc_tiling_on_sc=False)


def _flatten_body(in_ref, out_ref):
    out_ref[...] = in_ref[...].reshape(LATENT_DIM, FLAT_TILES, LANES)


@jax.jit
def _flatten(embT):
    return pl.pallas_call(
        _flatten_body,
        grid=(FLAT_GRID,),
        in_specs=[pl.BlockSpec((LATENT_DIM, FLAT_TILES * LANES), lambda j: (0, j))],
        out_specs=pl.BlockSpec((LATENT_DIM, FLAT_TILES, LANES), lambda j: (0, j, 0)),
        out_shape=jax.ShapeDtypeStruct((LATENT_DIM, SLAB_TILES, LANES), jnp.float32),
        compiler_params=pltpu.CompilerParams(
            dimension_semantics=("arbitrary",)),
    )(embT)


def _fire_gathers(flat_hbm, idx_v, gath_v, sem):
    descs = []
    for d in range(LATENT_DIM):
        stripe = flat_hbm.at[pl.ds(d * SLAB, SLAB)]
        for j in range(NUM_CHUNKS):
            dst = pl.ds(d * ROWS_PER_WORKER + j * IDX_CHUNK, IDX_CHUNK)
            descs.append(pltpu.async_copy(
                stripe.at[idx_v.at[j]], gath_v.at[dst], sem))
    return descs


def _gather_u_body(uidx_hbm, uflat_hbm, ug_hbm, uidx_v, u_gath, sem):
    wid = lax.axis_index("c") * NUM_SUBCORES + lax.axis_index("s")
    pltpu.sync_copy(uidx_hbm.at[wid], uidx_v)
    for dsc in _fire_gathers(uflat_hbm, uidx_v, u_gath, sem):
        dsc.wait()
    pltpu.sync_copy(u_gath, ug_hbm.at[pl.ds(wid * GATH, GATH)])


@jax.jit
def _gather_u(uidx3, uflat):
    mesh = plsc.VectorSubcoreMesh(core_axis_name="c", subcore_axis_name="s")
    f = functools.partial(
        pl.kernel,
        mesh=mesh,
        out_type=jax.ShapeDtypeStruct((NUM_WORKERS * GATH,), jnp.float32),
        compiler_params=_SC_PARAMS,
        scratch_types=[
            pltpu.VMEM((NUM_CHUNKS, IDX_CHUNK), jnp.int32),
            pltpu.VMEM((GATH,), jnp.float32),
            pltpu.SemaphoreType.DMA,
        ],
    )(_gather_u_body)
    return f(uidx3, uflat)


def _gmf_body(iidx_hbm, iflat_hbm, ug_hbm, w_hbm, b_hbm,
              out_hbm, iidx_v, u_gath, i_gath, w_v, b_v, out_v, sem):
    wid = lax.axis_index("c") * NUM_SUBCORES + lax.axis_index("s")
    pltpu.sync_copy(iidx_hbm.at[wid], iidx_v)
    descs = _fire_gathers(iflat_hbm, iidx_v, i_gath, sem)
    pltpu.sync_copy(ug_hbm.at[pl.ds(wid * GATH, GATH)], u_gath)
    pltpu.sync_copy(w_hbm, w_v)
    pltpu.sync_copy(b_hbm, b_v)
    for dsc in descs:
        dsc.wait()

    b_reg = b_v[...]
    w_reg = w_v[...]

    def group(g, carry):
        sl = pl.ds(g * 16, 16)
        acc = jnp.zeros((16,), jnp.float32)
        for d in range(LATENT_DIM):
            dsl = pl.ds(d * ROWS_PER_WORKER + g * 16, 16)
            acc = acc + u_gath[dsl] * i_gath[dsl] * w_reg[d]
        logits = acc + b_reg
        rating = 1.0 / (1.0 + jnp.exp(-logits))
        out_v[sl] = rating
        return carry

    lax.fori_loop(0, GROUPS, group, 0)

    pltpu.sync_copy(out_v, out_hbm.at[pl.ds(wid * ROWS_PER_WORKER, ROWS_PER_WORKER)])


@jax.jit
def _gmf(iidx3, iflat, ug, w16, b16):
    mesh = plsc.VectorSubcoreMesh(core_axis_name="c", subcore_axis_name="s")
    f = functools.partial(
        pl.kernel,
        mesh=mesh,
        out_type=jax.ShapeDtypeStruct((BATCH,), jnp.float32),
        compiler_params=_SC_PARAMS,
        scratch_types=[
            pltpu.VMEM((NUM_CHUNKS, IDX_CHUNK), jnp.int32),
            pltpu.VMEM((GATH,), jnp.float32),
            pltpu.VMEM((GATH,), jnp.float32),
            pltpu.VMEM((LATENT_DIM,), jnp.float32),
            pltpu.VMEM((LATENT_DIM,), jnp.float32),
            pltpu.VMEM((ROWS_PER_WORKER,), jnp.float32),
            pltpu.SemaphoreType.DMA,
        ],
    )(_gmf_body)
    return f(iidx3, iflat, ug, w16, b16)


def kernel(user_indices, item_indices, domain_idc, embedding_user,
           embedding_item, affine_W, affine_b):
    del domain_idc
    uidx3 = user_indices.reshape(NUM_WORKERS, NUM_CHUNKS, IDX_CHUNK)
    iidx3 = item_indices.reshape(NUM_WORKERS, NUM_CHUNKS, IDX_CHUNK)
    uflat = _flatten(embedding_user.T).reshape(LATENT_DIM * SLAB)
    ug = _gather_u(uidx3, uflat)
    iflat = _flatten(embedding_item.T).reshape(LATENT_DIM * SLAB)
    w16 = affine_W.reshape(LATENT_DIM)
    b16 = jnp.broadcast_to(affine_b, (LATENT_DIM,))
    out = _gmf(iidx3, iflat, ug, w16, b16)
    return out.reshape(BATCH, 1)
